# baseline (speedup 1.0000x reference)
import functools

import jax
import jax.numpy as jnp
from jax import lax
from jax.experimental import pallas as pl
from jax.experimental.pallas import tpu as pltpu
from jax.experimental.pallas import tpu_sc as plsc

_D = 64
_B_TOTAL = 4096 * 26
_NC, _NS = 2, 16
_NW = _NC * _NS
_B_PER_W = _B_TOTAL // _NW
_CHUNK = 416
_NCHUNK = _B_PER_W // _CHUNK
_NBUF = 4

_mesh = plsc.VectorSubcoreMesh(core_axis_name="c", subcore_axis_name="s")


@functools.partial(
    pl.kernel,
    mesh=_mesh,
    out_type=jax.ShapeDtypeStruct((_B_TOTAL, _D), jnp.float32),
    scratch_types=[
        pltpu.VMEM((_B_PER_W,), jnp.int32),
        pltpu.VMEM((_NBUF, _CHUNK, _D), jnp.float32),
        pltpu.SemaphoreType.DMA,
        pltpu.SemaphoreType.DMA,
    ],
    compiler_params=pltpu.CompilerParams(use_tc_tiling_on_sc=False),
)
def _gather_rows(table_hbm, idx_hbm, out_hbm, idx_v, rows_v, sem_g, sem_o):
    wid = lax.axis_index("s") * _NC + lax.axis_index("c")
    base = wid * _B_PER_W
    pltpu.sync_copy(idx_hbm.at[pl.ds(base, _B_PER_W)], idx_v)

    def gather_start(ci):
        return pltpu.async_copy(
            table_hbm.at[idx_v.at[pl.ds(ci * _CHUNK, _CHUNK)]],
            rows_v.at[ci % _NBUF],
            sem_g,
        )

    def out_start(ci):
        return pltpu.async_copy(
            rows_v.at[ci % _NBUF],
            out_hbm.at[pl.ds(base + ci * _CHUNK, _CHUNK)],
            sem_o,
        )

    gathers = [gather_start(0)]
    outs = []
    for ci in range(_NCHUNK):
        gathers[ci].wait()
        if ci + 1 < _NCHUNK:
            if ci + 1 >= _NBUF:
                outs[ci + 1 - _NBUF].wait()
            gathers.append(gather_start(ci + 1))
        outs.append(out_start(ci))
    for ci in range(max(0, _NCHUNK - _NBUF), _NCHUNK):
        outs[ci].wait()


@jax.jit
def kernel(x, embed):
    flat = x.reshape(-1).astype(jnp.int32)
    out = _gather_rows(embed, flat)
    return out.reshape(x.shape[0], x.shape[1], _D)


# speedup vs baseline: 1.0296x; 1.0296x over previous
import functools

import jax
import jax.numpy as jnp
from jax import lax
from jax.experimental import pallas as pl
from jax.experimental.pallas import tpu as pltpu
from jax.experimental.pallas import tpu_sc as plsc

_D = 64
_DP = 128
_B = 4096
_F = 26
_NC, _NS = 2, 16
_NW = _NC * _NS
_BW = _B // _NW
_NB = 8
_NG = _BW // _NB
_RING = 4

_mesh = plsc.VectorSubcoreMesh(core_axis_name="c", subcore_axis_name="s")


@functools.partial(
    pl.kernel,
    mesh=_mesh,
    out_type=jax.ShapeDtypeStruct((_B, _F, _D), jnp.float32),
    scratch_types=[
        pltpu.VMEM((_BW, _F), jnp.int32),
        pltpu.VMEM((_RING, _F, _DP), jnp.float32),
        pltpu.VMEM((_NB, _F, _D), jnp.float32),
        pltpu.SemaphoreType.DMA,
    ],
    compiler_params=pltpu.CompilerParams(use_tc_tiling_on_sc=True),
)
def _gather_rows(table_hbm, idx_hbm, out_hbm, idx_v, gbuf, cbuf, sem):
    wid = lax.axis_index("s") * _NC + lax.axis_index("c")
    pltpu.sync_copy(idx_hbm.at[pl.ds(wid * _BW, _BW)], idx_v)

    def gather(i, slot):
        return pltpu.make_async_copy(
            table_hbm.at[idx_v.at[i]], gbuf.at[slot], sem
        )

    def compact(slot, kb):
        def row(r, carry):
            for k in range(_D // 16):
                cbuf[kb, r, pl.ds(k * 16, 16)] = gbuf[slot, r, pl.ds(k * 16, 16)]
            return carry

        lax.fori_loop(0, _F, row, 0)

    for p in range(_RING):
        gather(p, p).start()

    def loop_body(ch, carry):
        for kb in range(_NB):
            i = ch * _NB + kb
            slot = kb % _RING
            gather(i, slot).wait()
            compact(slot, kb)
            gather(i + _RING, slot).start()
        pltpu.sync_copy(cbuf, out_hbm.at[pl.ds(wid * _BW + ch * _NB, _NB)])
        return carry

    lax.fori_loop(0, _NG - 1, loop_body, 0)

    ch = _NG - 1
    for kb in range(_NB):
        i = ch * _NB + kb
        slot = kb % _RING
        gather(i, slot).wait()
        compact(slot, kb)
        if i + _RING < _BW:
            gather(i + _RING, slot).start()
    pltpu.sync_copy(cbuf, out_hbm.at[pl.ds(wid * _BW + ch * _NB, _NB)])


@jax.jit
def kernel(x, embed):
    padded = jnp.concatenate(
        [embed, jnp.zeros((embed.shape[0], _DP - _D), jnp.float32)], axis=1
    )
    return _gather_rows(padded, x.astype(jnp.int32))
